# baseline (speedup 1.0000x reference)
import jax
import jax.numpy as jnp
from jax import lax
from jax.experimental import pallas as pl
from jax.experimental.pallas import tpu as pltpu

_K = 128
_STAGES = 10
_BB = 2
_UNROLL = 5


def _dfc_body(x_ref, p0_ref, cluster_ref, proto_ref, sim_ref):
    n = x_ref.shape[2]
    xc = [x_ref[j] for j in range(_BB)]
    xt = [jnp.transpose(a) for a in xc]

    def stage(_, carry):
        ps = list(carry[:_BB])
        ss = list(carry[_BB:])
        for _ in range(_UNROLL):
            for j in range(_BB):
                p = ps[j]
                sim = jnp.dot(xt[j], p, preferred_element_type=jnp.float32)
                m = jnp.max(sim, axis=-1, keepdims=True)
                e = jnp.exp(sim - m)
                s = e / jnp.sum(e, axis=-1, keepdims=True)
                w = s / jnp.sum(s, axis=0, keepdims=True)
                ps[j] = p * 0.5 + jnp.dot(xc[j], w,
                                          preferred_element_type=jnp.float32) * 0.5
                ss[j] = s
        return tuple(ps) + tuple(ss)

    init = tuple(p0_ref[j] for j in range(_BB))
    init += tuple(jnp.zeros((n, _K), jnp.float32) for _ in range(_BB))
    res = lax.fori_loop(0, _STAGES // _UNROLL, stage, init)

    for j in range(_BB):
        p, s = res[j], res[_BB + j]
        proto_ref[j] = jnp.transpose(p)
        sim_ref[j] = s
        m = jnp.max(s, axis=-1, keepdims=True)
        idx = lax.broadcasted_iota(jnp.int32, (n, _K), 1)
        cluster_ref[j, 0] = jnp.min(jnp.where(s == m, idx, _K), axis=-1)


def kernel(x):
    b, c, n = x.shape
    p0 = x[:, :, :: n // _K]

    cluster3, proto, sim = pl.pallas_call(
        _dfc_body,
        grid=(b // _BB,),
        in_specs=[
            pl.BlockSpec((_BB, c, n), lambda i: (i, 0, 0)),
            pl.BlockSpec((_BB, c, _K), lambda i: (i, 0, 0)),
        ],
        out_specs=[
            pl.BlockSpec((_BB, 1, n), lambda i: (i, 0, 0)),
            pl.BlockSpec((_BB, _K, c), lambda i: (i, 0, 0)),
            pl.BlockSpec((_BB, n, _K), lambda i: (i, 0, 0)),
        ],
        out_shape=[
            jax.ShapeDtypeStruct((b, 1, n), jnp.int32),
            jax.ShapeDtypeStruct((b, _K, c), jnp.float32),
            jax.ShapeDtypeStruct((b, n, _K), jnp.float32),
        ],
    )(x, p0)

    return cluster3.reshape(b, n), proto, sim


# speedup vs baseline: 1.0865x; 1.0865x over previous
import jax
import jax.numpy as jnp
from jax import lax
from jax.experimental import pallas as pl
from jax.experimental.pallas import tpu as pltpu

_K = 128
_STAGES = 10
_BB = 4
_UNROLL = 5


def _dfc_body(x_ref, p0_ref, cluster_ref, proto_ref, sim_ref):
    n = x_ref.shape[2]
    xc = [x_ref[j] for j in range(_BB)]
    xt = [jnp.transpose(a) for a in xc]

    def stage(_, carry):
        ps = list(carry[:_BB])
        ss = list(carry[_BB:])
        for _ in range(_UNROLL):
            for j in range(_BB):
                p = ps[j]
                sim = jnp.dot(xt[j], p, preferred_element_type=jnp.float32)
                m = jnp.max(sim, axis=-1, keepdims=True)
                e = jnp.exp(sim - m)
                s = e / jnp.sum(e, axis=-1, keepdims=True)
                w = s / jnp.sum(s, axis=0, keepdims=True)
                ps[j] = p * 0.5 + jnp.dot(xc[j], w,
                                          preferred_element_type=jnp.float32) * 0.5
                ss[j] = s
        return tuple(ps) + tuple(ss)

    init = tuple(p0_ref[j] for j in range(_BB))
    init += tuple(jnp.zeros((n, _K), jnp.float32) for _ in range(_BB))
    res = lax.fori_loop(0, _STAGES // _UNROLL, stage, init)

    for j in range(_BB):
        p, s = res[j], res[_BB + j]
        proto_ref[j] = jnp.transpose(p)
        sim_ref[j] = s
        m = jnp.max(s, axis=-1, keepdims=True)
        idx = lax.broadcasted_iota(jnp.int32, (n, _K), 1)
        cluster_ref[j, 0] = jnp.min(jnp.where(s == m, idx, _K), axis=-1)


def kernel(x):
    b, c, n = x.shape
    p0 = x[:, :, :: n // _K]

    cluster3, proto, sim = pl.pallas_call(
        _dfc_body,
        grid=(b // _BB,),
        in_specs=[
            pl.BlockSpec((_BB, c, n), lambda i: (i, 0, 0)),
            pl.BlockSpec((_BB, c, _K), lambda i: (i, 0, 0)),
        ],
        out_specs=[
            pl.BlockSpec((_BB, 1, n), lambda i: (i, 0, 0)),
            pl.BlockSpec((_BB, _K, c), lambda i: (i, 0, 0)),
            pl.BlockSpec((_BB, n, _K), lambda i: (i, 0, 0)),
        ],
        out_shape=[
            jax.ShapeDtypeStruct((b, 1, n), jnp.int32),
            jax.ShapeDtypeStruct((b, _K, c), jnp.float32),
            jax.ShapeDtypeStruct((b, n, _K), jnp.float32),
        ],
        compiler_params=pltpu.CompilerParams(
            vmem_limit_bytes=64 * 1024 * 1024),
    )(x, p0)

    return cluster3.reshape(b, n), proto, sim


# speedup vs baseline: 2.5680x; 2.3635x over previous
import jax
import jax.numpy as jnp
from jax import lax
from jax.experimental import pallas as pl
from jax.experimental.pallas import tpu as pltpu

_K = 128
_STAGES = 10
_BB = 4
_UNROLL = 5


def _dfc_body(x_ref, cluster_ref, proto_ref, sim_ref):
    n = x_ref.shape[2]
    c = x_ref.shape[1]
    xc = [x_ref[j] for j in range(_BB)]
    xt = [jnp.transpose(a) for a in xc]
    p0 = [jnp.transpose(jnp.reshape(a, (_K, n // _K, c))[:, 0, :])
          for a in xt]

    def stage(_, carry):
        ps = list(carry[:_BB])
        ss = list(carry[_BB:])
        for _ in range(_UNROLL):
            for j in range(_BB):
                p = ps[j]
                sim = jnp.dot(xt[j], p, preferred_element_type=jnp.float32)
                m = jnp.max(sim, axis=-1, keepdims=True)
                e = jnp.exp(sim - m)
                s = e / jnp.sum(e, axis=-1, keepdims=True)
                w = s / jnp.sum(s, axis=0, keepdims=True)
                ps[j] = p * 0.5 + jnp.dot(xc[j], w,
                                          preferred_element_type=jnp.float32) * 0.5
                ss[j] = s
        return tuple(ps) + tuple(ss)

    init = tuple(p0)
    init += tuple(jnp.zeros((n, _K), jnp.float32) for _ in range(_BB))
    res = lax.fori_loop(0, _STAGES // _UNROLL, stage, init)

    for j in range(_BB):
        p, s = res[j], res[_BB + j]
        proto_ref[j] = jnp.transpose(p)
        sim_ref[j] = s
        m = jnp.max(s, axis=-1, keepdims=True)
        idx = lax.broadcasted_iota(jnp.int32, (n, _K), 1)
        cluster_ref[j, 0] = jnp.min(jnp.where(s == m, idx, _K), axis=-1)


def kernel(x):
    b, c, n = x.shape

    cluster3, proto, sim = pl.pallas_call(
        _dfc_body,
        grid=(b // _BB,),
        in_specs=[
            pl.BlockSpec((_BB, c, n), lambda i: (i, 0, 0)),
        ],
        out_specs=[
            pl.BlockSpec((_BB, 1, n), lambda i: (i, 0, 0)),
            pl.BlockSpec((_BB, _K, c), lambda i: (i, 0, 0)),
            pl.BlockSpec((_BB, n, _K), lambda i: (i, 0, 0)),
        ],
        out_shape=[
            jax.ShapeDtypeStruct((b, 1, n), jnp.int32),
            jax.ShapeDtypeStruct((b, _K, c), jnp.float32),
            jax.ShapeDtypeStruct((b, n, _K), jnp.float32),
        ],
        compiler_params=pltpu.CompilerParams(
            vmem_limit_bytes=64 * 1024 * 1024),
    )(x)

    return cluster3.reshape(b, n), proto, sim


# speedup vs baseline: 2.6274x; 1.0231x over previous
import jax
import jax.numpy as jnp
from jax import lax
from jax.experimental import pallas as pl
from jax.experimental.pallas import tpu as pltpu

_K = 128
_STAGES = 10
_BB = 4
_UNROLL = 5


def _dfc_body(x_ref, cluster_ref, proto_ref, sim_ref):
    n = x_ref.shape[2]
    c = x_ref.shape[1]
    xc = [x_ref[j] for j in range(_BB)]
    xt = [jnp.transpose(a) for a in xc]
    p0 = [jnp.transpose(jnp.reshape(a, (_K, n // _K, c))[:, 0, :])
          for a in xt]

    def stage(_, carry):
        ps = list(carry[:_BB])
        ss = list(carry[_BB:])
        for _ in range(_UNROLL):
            for j in range(_BB):
                p = ps[j]
                sim = jnp.dot(xt[j], p, preferred_element_type=jnp.float32)
                m = jnp.max(sim, axis=-1, keepdims=True)
                e = jnp.exp(sim - m)
                s = e / jnp.sum(e, axis=-1, keepdims=True)
                w = s / jnp.sum(s, axis=0, keepdims=True)
                ps[j] = p * 0.5 + jnp.dot(xc[j], w,
                                          preferred_element_type=jnp.float32) * 0.5
                ss[j] = s
        return tuple(ps) + tuple(ss)

    init = tuple(p0)
    init += tuple(jnp.zeros((n, _K), jnp.float32) for _ in range(_BB))
    res = lax.fori_loop(0, _STAGES // _UNROLL, stage, init)

    for j in range(_BB):
        p, s = res[j], res[_BB + j]
        proto_ref[j] = jnp.transpose(p)
        sim_ref[j] = s
        cluster_ref[j, 0] = jnp.argmax(s, axis=-1).astype(jnp.int32)


def kernel(x):
    b, c, n = x.shape

    cluster3, proto, sim = pl.pallas_call(
        _dfc_body,
        grid=(b // _BB,),
        in_specs=[
            pl.BlockSpec((_BB, c, n), lambda i: (i, 0, 0)),
        ],
        out_specs=[
            pl.BlockSpec((_BB, 1, n), lambda i: (i, 0, 0)),
            pl.BlockSpec((_BB, _K, c), lambda i: (i, 0, 0)),
            pl.BlockSpec((_BB, n, _K), lambda i: (i, 0, 0)),
        ],
        out_shape=[
            jax.ShapeDtypeStruct((b, 1, n), jnp.int32),
            jax.ShapeDtypeStruct((b, _K, c), jnp.float32),
            jax.ShapeDtypeStruct((b, n, _K), jnp.float32),
        ],
        compiler_params=pltpu.CompilerParams(
            vmem_limit_bytes=64 * 1024 * 1024),
    )(x)

    return cluster3.reshape(b, n), proto, sim
